# Initial kernel scaffold; baseline (speedup 1.0000x reference)
import jax
import jax.numpy as jnp
from jax.experimental import pallas as pl


def kernel(x, edge_index, edge_attr, velocity, diff_param):
    raise NotImplementedError("write your pallas kernel here")


# speedup vs baseline: 28.8405x; 28.8405x over previous
import functools

import jax
import jax.numpy as jnp
from jax import lax
from jax.experimental import pallas as pl
from jax.experimental.pallas import tpu as pltpu
from jax.experimental.pallas import tpu_sc as plsc

N = 100000
E = 1600000
T = 4
NC = 2
NS = 16
L = 16
NW = NC * NS
EPW = E // NW
W = 10000
NWIN = EPW // W
NCHUNK = W // L
NPAD = 100352
SLICE = NPAD // NS


def _sc_body(x0, x1, x2, x3, src, dst, ea0, ea1, pv0, pv1, pdp, out,
             v0v, v1v, dpv, src_v, dst_v, ea0_v, ea1_v, wm_v,
             xg0, xg1, xg2, xg3,
             acc0, acc1, acc2, acc3, dega,
             g0, g1, g2, g3, s0, s1, s2, s3, s4, l0, l1, l2, l3):
    gsem = (g0, g1, g2, g3)
    ssem = (s0, s1, s2, s3, s4)
    c = lax.axis_index("c")
    s = lax.axis_index("s")
    wid = s * NC + c
    xh = (x0, x1, x2, x3)
    xg = (xg0, xg1, xg2, xg3)
    accs = (acc0, acc1, acc2, acc3, dega)

    pltpu.sync_copy(pv0, v0v)
    pltpu.sync_copy(pv1, v1v)
    pltpu.sync_copy(pdp, dpv)
    v0 = v0v[...]
    v1 = v1v[...]
    dp = dpv[...]
    d2 = dp * dp

    def zloop(j, carry):
        ea0_v[pl.ds(j * L, L)] = jnp.zeros((L,), jnp.float32)
        return carry
    lax.fori_loop(0, SLICE // L, zloop, 0)
    off = pl.multiple_of(s * SLICE, 8)
    for a in accs:
        pltpu.sync_copy(ea0_v.at[pl.ds(0, SLICE)], a.at[pl.ds(off, SLICE)])
    plsc.subcore_barrier()

    def window(w, carry):
        base = pl.multiple_of(wid * EPW + w * W, 8)
        cs = pltpu.async_copy(src.at[pl.ds(base, W)], src_v, l0)
        cd = pltpu.async_copy(dst.at[pl.ds(base, W)], dst_v, l1)
        c0 = pltpu.async_copy(ea0.at[pl.ds(base, W)], ea0_v, l2)
        c1 = pltpu.async_copy(ea1.at[pl.ds(base, W)], ea1_v, l3)
        cd.wait()
        gds = [pltpu.async_copy(xh[t].at[dst_v], xg[t], gsem[t])
               for t in range(T)]
        c0.wait()
        c1.wait()
        cs.wait()
        for g in gds:
            g.wait()

        def chunk(i, carry):
            sl = pl.ds(i * L, L)
            a = ea0_v[sl]
            b = ea1_v[sl]
            coef = (a * v0 + b * v1) * -0.5
            wm_v[sl] = coef - d2
            p = coef + d2
            for t in range(T):
                xg[t][sl] = p * xg[t][sl]
            return carry
        lax.fori_loop(0, NCHUNK, chunk, 0)

        scs = [pltpu.async_copy(xg[t], accs[t].at[src_v], ssem[t],
                                add=True)
               for t in range(T)]
        scs.append(pltpu.async_copy(wm_v, dega.at[src_v], ssem[4],
                                    add=True))
        for d in scs:
            d.wait()
        return carry
    lax.fori_loop(0, NWIN, window, 0)

    plsc.subcore_barrier()
    for i, a in enumerate(accs):
        oo = pl.multiple_of((c * 5 + i) * NPAD + off, 8)
        pltpu.sync_copy(a.at[pl.ds(off, SLICE)], out.at[pl.ds(oo, SLICE)])


_sc_call = functools.partial(
    pl.kernel,
    out_type=jax.ShapeDtypeStruct((NC * 5 * NPAD,), jnp.float32),
    mesh=plsc.VectorSubcoreMesh(core_axis_name="c", subcore_axis_name="s",
                                num_cores=NC, num_subcores=NS),
    compiler_params=pltpu.CompilerParams(needs_layout_passes=False),
    scratch_types=[
        pltpu.VMEM((L,), jnp.float32),
        pltpu.VMEM((L,), jnp.float32),
        pltpu.VMEM((L,), jnp.float32),
        pltpu.VMEM((W,), jnp.int32),
        pltpu.VMEM((W,), jnp.int32),
        pltpu.VMEM((W,), jnp.float32),
        pltpu.VMEM((W,), jnp.float32),
        pltpu.VMEM((W,), jnp.float32),
        pltpu.VMEM((W,), jnp.float32),
        pltpu.VMEM((W,), jnp.float32),
        pltpu.VMEM((W,), jnp.float32),
        pltpu.VMEM((W,), jnp.float32),
        pltpu.VMEM_SHARED((NPAD,), jnp.float32),
        pltpu.VMEM_SHARED((NPAD,), jnp.float32),
        pltpu.VMEM_SHARED((NPAD,), jnp.float32),
        pltpu.VMEM_SHARED((NPAD,), jnp.float32),
        pltpu.VMEM_SHARED((NPAD,), jnp.float32),
    ] + [pltpu.SemaphoreType.DMA] * 13,
)(_sc_body)


BN = 6272


def _combine_body(x_ref, acc_ref, o_ref):
    a = acc_ref[0] + acc_ref[1]
    deg = a[4:5, :]
    o_ref[...] = x_ref[...] * (1.0 + deg) + a[0:4, :]


def _combine(x4, accs):
    return pl.pallas_call(
        _combine_body,
        out_shape=jax.ShapeDtypeStruct((T, N), jnp.float32),
        grid=(NPAD // BN,),
        in_specs=[
            pl.BlockSpec((T, BN), lambda j: (0, j)),
            pl.BlockSpec((NC, 5, BN), lambda j: (0, 0, j)),
        ],
        out_specs=pl.BlockSpec((T, BN), lambda j: (0, j)),
    )(x4, accs)


@jax.jit
def kernel(x, edge_index, edge_attr, velocity, diff_param):
    x4 = x.reshape(T, N)
    src = edge_index[0].astype(jnp.int32)
    dst = edge_index[1].astype(jnp.int32)
    ea0 = edge_attr[:, 0]
    ea1 = edge_attr[:, 1]
    pv0 = jnp.full((L,), velocity[0], jnp.float32)
    pv1 = jnp.full((L,), velocity[1], jnp.float32)
    pdp = jnp.full((L,), diff_param[0], jnp.float32)
    accs = _sc_call(x4[0], x4[1], x4[2], x4[3], src, dst, ea0, ea1,
                    pv0, pv1, pdp)
    out4 = _combine(x4, accs.reshape(NC, 5, NPAD))
    return out4.reshape(x.shape)
